# baseline (speedup 1.0000x reference)
import functools

import jax
import jax.numpy as jnp
from jax import lax
from jax.experimental import pallas as pl
from jax.experimental.pallas import tpu as pltpu
from jax.experimental.pallas import tpu_sc as plsc

NC = 2
NS = 16
NW = NC * NS
IDX_MINOR = 128


def _gather_body(n_rows, rows_per_chunk, d, w_hbm, xw_hbm, out_hbm,
                 idx_v, rows_v, sem_in):
    wid = lax.axis_index("s") * NC + lax.axis_index("c")
    b_per_w = n_rows * IDX_MINOR
    chunk = rows_per_chunk * IDX_MINOR
    n_chunks = n_rows // rows_per_chunk

    pltpu.sync_copy(xw_hbm.at[wid], idx_v)

    @pl.loop(0, n_chunks)
    def _chunk(g):
        row0 = g * rows_per_chunk
        cps = []
        for j in range(rows_per_chunk):
            cps.append(pltpu.async_copy(
                w_hbm.at[idx_v.at[row0 + j]],
                rows_v.at[pl.ds(j * IDX_MINOR, IDX_MINOR)],
                sem_in))
        for cp in cps:
            cp.wait()
        pltpu.sync_copy(
            rows_v, out_hbm.at[pl.ds(wid * b_per_w + g * chunk, chunk)])


def _embedding_gather(x_flat, W):
    V, D = W.shape
    B_total = x_flat.shape[0]
    assert B_total % (NW * IDX_MINOR) == 0
    n_rows = B_total // (NW * IDX_MINOR)
    rows_per_chunk = 4
    chunk = rows_per_chunk * IDX_MINOR

    xw = x_flat.reshape(NW, n_rows, IDX_MINOR)
    mesh = plsc.VectorSubcoreMesh(
        core_axis_name="c", subcore_axis_name="s",
        num_cores=NC, num_subcores=NS)

    body = functools.partial(_gather_body, n_rows, rows_per_chunk, D)
    return pl.kernel(
        body,
        out_type=jax.ShapeDtypeStruct((B_total, D), W.dtype),
        mesh=mesh,
        scratch_types=[
            pltpu.VMEM((n_rows, IDX_MINOR), jnp.int32),
            pltpu.VMEM((chunk, D), W.dtype),
            pltpu.SemaphoreType.DMA,
        ],
        compiler_params=pltpu.CompilerParams(use_tc_tiling_on_sc=False),
    )(W, xw)


def kernel(x, target, text_inputs, W):
    B, L = x.shape
    D = W.shape[1]
    out = _embedding_gather(x.reshape(B * L), W)
    return out.reshape(B, L, D)


# speedup vs baseline: 1.0208x; 1.0208x over previous
import functools

import jax
import jax.numpy as jnp
from jax import lax
from jax.experimental import pallas as pl
from jax.experimental.pallas import tpu as pltpu
from jax.experimental.pallas import tpu_sc as plsc

NC = 2
NS = 16
NW = NC * NS
IDX_MINOR = 128


NBUF = 3


def _gather_body(n_rows, rows_per_chunk, d, w_hbm, xw_hbm, out_hbm,
                 idx_v, rows_v, sem_g, sem_o):
    wid = lax.axis_index("s") * NC + lax.axis_index("c")
    b_per_w = n_rows * IDX_MINOR
    chunk = rows_per_chunk * IDX_MINOR
    n_chunks = n_rows // rows_per_chunk

    pltpu.sync_copy(xw_hbm.at[wid], idx_v)

    def fire_gathers(c, b):
        row0 = c * rows_per_chunk
        for j in range(rows_per_chunk):
            pltpu.async_copy(
                w_hbm.at[idx_v.at[row0 + j]],
                rows_v.at[b, pl.ds(j * IDX_MINOR, IDX_MINOR)],
                sem_g.at[b])

    def wait_gathers(b):
        pltpu.make_async_copy(
            w_hbm.at[pl.ds(0, chunk)], rows_v.at[b], sem_g.at[b]).wait()

    def wait_out(b):
        pltpu.make_async_copy(
            rows_v.at[b], out_hbm.at[pl.ds(0, chunk)], sem_o.at[b]).wait()

    fire_gathers(0, 0)

    @pl.loop(0, n_chunks)
    def _chunk(c):
        b = lax.rem(c, NBUF)
        wait_gathers(b)
        pltpu.async_copy(
            rows_v.at[b],
            out_hbm.at[pl.ds(wid * b_per_w + c * chunk, chunk)],
            sem_o.at[b])
        nb = lax.rem(c + 1, NBUF)

        @pl.when(c + 1 - NBUF >= 0)
        def _():
            wait_out(nb)

        @pl.when(c + 1 < n_chunks)
        def _():
            fire_gathers(c + 1, nb)

    for c in range(n_chunks - NBUF + 1, n_chunks):
        wait_out(c % NBUF)


def _embedding_gather(x_flat, W):
    V, D = W.shape
    B_total = x_flat.shape[0]
    assert B_total % (NW * IDX_MINOR) == 0
    n_rows = B_total // (NW * IDX_MINOR)
    rows_per_chunk = 4
    chunk = rows_per_chunk * IDX_MINOR

    xw = x_flat.reshape(NW, n_rows, IDX_MINOR)
    mesh = plsc.VectorSubcoreMesh(
        core_axis_name="c", subcore_axis_name="s",
        num_cores=NC, num_subcores=NS)

    body = functools.partial(_gather_body, n_rows, rows_per_chunk, D)
    return pl.kernel(
        body,
        out_type=jax.ShapeDtypeStruct((B_total, D), W.dtype),
        mesh=mesh,
        scratch_types=[
            pltpu.VMEM((n_rows, IDX_MINOR), jnp.int32),
            pltpu.VMEM((NBUF, chunk, D), W.dtype),
            pltpu.SemaphoreType.DMA((NBUF,)),
            pltpu.SemaphoreType.DMA((NBUF,)),
        ],
        compiler_params=pltpu.CompilerParams(use_tc_tiling_on_sc=False),
    )(W, xw)


def kernel(x, target, text_inputs, W):
    B, L = x.shape
    D = W.shape[1]
    out = _embedding_gather(x.reshape(B * L), W)
    return out.reshape(B, L, D)


# speedup vs baseline: 1.0218x; 1.0010x over previous
import functools

import jax
import jax.numpy as jnp
from jax import lax
from jax.experimental import pallas as pl
from jax.experimental.pallas import tpu as pltpu
from jax.experimental.pallas import tpu_sc as plsc

NC = 2
NS = 16
NW = NC * NS
NBUF = 3
ROWS_PER_CHUNK = 2


def _gather_body(b_rows, seq, d, w_hbm, x_hbm, out_hbm,
                 idx_v, rows_v, sem_g, sem_o):
    wid = lax.axis_index("s") * NC + lax.axis_index("c")
    rows_per_w = b_rows // NW
    n_chunks = rows_per_w // ROWS_PER_CHUNK
    row0_w = wid * rows_per_w
    splits = [(0, 128), (128, seq - 128)] if seq > 128 else [(0, seq)]

    pltpu.sync_copy(x_hbm.at[pl.ds(row0_w, rows_per_w)], idx_v)

    def fire_gathers(c, b):
        for r in range(ROWS_PER_CHUNK):
            for (off, ln) in splits:
                pltpu.async_copy(
                    w_hbm.at[idx_v.at[c * ROWS_PER_CHUNK + r, pl.ds(off, ln)]],
                    rows_v.at[b, r, pl.ds(off, ln)],
                    sem_g.at[b])

    def wait_gathers(b):
        pltpu.make_async_copy(
            out_hbm.at[pl.ds(0, ROWS_PER_CHUNK)], rows_v.at[b],
            sem_g.at[b]).wait()

    def wait_out(b):
        pltpu.make_async_copy(
            rows_v.at[b], out_hbm.at[pl.ds(0, ROWS_PER_CHUNK)],
            sem_o.at[b]).wait()

    fire_gathers(0, 0)

    @pl.loop(0, n_chunks)
    def _chunk(c):
        b = lax.rem(c, NBUF)
        wait_gathers(b)
        pltpu.async_copy(
            rows_v.at[b],
            out_hbm.at[pl.ds(row0_w + c * ROWS_PER_CHUNK, ROWS_PER_CHUNK)],
            sem_o.at[b])
        nb = lax.rem(c + 1, NBUF)

        @pl.when(c + 1 - NBUF >= 0)
        def _():
            wait_out(nb)

        @pl.when(c + 1 < n_chunks)
        def _():
            fire_gathers(c + 1, nb)

    for c in range(n_chunks - NBUF + 1, n_chunks):
        wait_out(c % NBUF)


def _embedding_gather(x, W):
    V, D = W.shape
    B, S = x.shape
    assert B % NW == 0 and (B // NW) % ROWS_PER_CHUNK == 0

    mesh = plsc.VectorSubcoreMesh(
        core_axis_name="c", subcore_axis_name="s",
        num_cores=NC, num_subcores=NS)

    body = functools.partial(_gather_body, B, S, D)
    return pl.kernel(
        body,
        out_type=jax.ShapeDtypeStruct((B, S, D), W.dtype),
        mesh=mesh,
        scratch_types=[
            pltpu.VMEM((B // NW, S), jnp.int32),
            pltpu.VMEM((NBUF, ROWS_PER_CHUNK, S, D), W.dtype),
            pltpu.SemaphoreType.DMA((NBUF,)),
            pltpu.SemaphoreType.DMA((NBUF,)),
        ],
        compiler_params=pltpu.CompilerParams(use_tc_tiling_on_sc=False),
    )(W, x)


def kernel(x, target, text_inputs, W):
    return _embedding_gather(x, W)
